# baseline (speedup 1.0000x reference)
import functools

import jax
import jax.numpy as jnp
from jax.experimental import pallas as pl
from jax.experimental.pallas import tpu as pltpu

_BLOCK_T = 512
_N_SLOTS = 4
_N_QUARTERS = 4


def _softmax_rows(logits):
    m = jnp.max(logits, axis=-1, keepdims=True)
    e = jnp.exp(logits - m)
    return e / jnp.sum(e, axis=-1, keepdims=True)


def _router_body(n_blocks, x_hbm, w_ref, b_ref, out_hbm,
                 x_vmem, o_vmem, in_sems, out_sems):
    n_slots = x_vmem.shape[0]
    block_t = x_vmem.shape[1]
    n_q = in_sems.shape[1]
    q_t = block_t // n_q

    def in_copy(blk, slot, q):
        return pltpu.make_async_copy(
            x_hbm.at[pl.ds(blk * block_t + q * q_t, q_t), :],
            x_vmem.at[slot, pl.ds(q * q_t, q_t), :],
            in_sems.at[slot, q])

    for slot in range(min(n_slots, n_blocks)):
        for q in range(n_q):
            in_copy(slot, slot, q).start()

    def step(i, carry):
        slot = jax.lax.rem(i, n_slots)
        for q in range(n_q):
            in_copy(i, slot, q).wait()

        logits = jax.lax.dot_general(
            x_vmem[slot].astype(jnp.bfloat16), w_ref[...],
            dimension_numbers=(((1,), (1,)), ((), ())),
            preferred_element_type=jnp.float32,
        ) + b_ref[...]
        o_vmem[pl.ds(i * block_t, block_t), :] = _softmax_rows(logits)

        @pl.when(i + n_slots < n_blocks)
        def _():
            for q in range(n_q):
                in_copy(i + n_slots, slot, q).start()

        return carry

    jax.lax.fori_loop(0, n_blocks, step, 0)

    out = pltpu.make_async_copy(o_vmem, out_hbm, out_sems)
    out.start()
    out.wait()


def kernel(x, W, b):
    n_tokens, hidden = x.shape
    n_experts = W.shape[0]
    block_t = min(_BLOCK_T, n_tokens)
    n_blocks = n_tokens // block_t
    n_slots = min(_N_SLOTS, n_blocks)
    n_q = _N_QUARTERS if block_t % _N_QUARTERS == 0 else 1
    return pl.pallas_call(
        functools.partial(_router_body, n_blocks),
        in_specs=[
            pl.BlockSpec(memory_space=pl.ANY),
            pl.BlockSpec(memory_space=pltpu.VMEM),
            pl.BlockSpec(memory_space=pltpu.VMEM),
        ],
        out_specs=pl.BlockSpec(memory_space=pl.ANY),
        out_shape=jax.ShapeDtypeStruct((n_tokens, n_experts), jnp.float32),
        compiler_params=pltpu.CompilerParams(
            skip_device_barrier=True,
            disable_bounds_checks=True,
        ),
        scratch_shapes=[
            pltpu.VMEM((n_slots, block_t, hidden), jnp.float32),
            pltpu.VMEM((n_tokens, n_experts), jnp.float32),
            pltpu.SemaphoreType.DMA((n_slots, n_q)),
            pltpu.SemaphoreType.DMA(()),
        ],
    )(x, W.astype(jnp.bfloat16), b.reshape(1, n_experts))


# speedup vs baseline: 1.0471x; 1.0471x over previous
import jax
import jax.numpy as jnp
from jax.experimental import pallas as pl
from jax.experimental.pallas import tpu as pltpu

_BLOCK_T = 1024


def _router_block(x_ref, w_ref, b_ref, out_ref):
    logits = jax.lax.dot_general(
        x_ref[...], w_ref[...],
        dimension_numbers=(((1,), (1,)), ((), ())),
        preferred_element_type=jnp.float32,
    )
    logits = logits + b_ref[...]
    m = jnp.max(logits, axis=-1, keepdims=True)
    e = jnp.exp(logits - m)
    out_ref[...] = e / jnp.sum(e, axis=-1, keepdims=True)


def kernel(x, W, b):
    n_tokens, hidden = x.shape
    n_experts = W.shape[0]
    block_t = min(_BLOCK_T, n_tokens)
    return pl.pallas_call(
        _router_block,
        grid=(n_tokens // block_t,),
        in_specs=[
            pl.BlockSpec((block_t, hidden), lambda i: (i, 0)),
            pl.BlockSpec((n_experts, hidden), lambda i: (0, 0)),
            pl.BlockSpec((1, n_experts), lambda i: (0, 0)),
        ],
        out_specs=pl.BlockSpec((block_t, n_experts), lambda i: (i, 0)),
        out_shape=jax.ShapeDtypeStruct((n_tokens, n_experts), jnp.float32),
        compiler_params=pltpu.CompilerParams(skip_device_barrier=True),
    )(x, W, b.reshape(1, n_experts))
